# baseline (speedup 1.0000x reference)
import functools

import jax
import jax.numpy as jnp
from jax import lax
from jax.experimental import pallas as pl
from jax.experimental.pallas import tpu as pltpu
from jax.experimental.pallas import tpu_sc as plsc


def _cycle_mlp_body(nblk5, xc_ref, xe_ref, xs_ref, wa_ref, m_ref, wc_ref,
                    b0_ref, w1_ref, b1_ref, w2_ref, b2_ref, out_ref):
    bf = jnp.bfloat16
    b0 = jnp.where(pl.program_id(0) < nblk5, b0_ref[0], b0_ref[1])
    h = (jnp.dot(xc_ref[...].astype(bf), wa_ref[...],
                 preferred_element_type=jnp.float32)
         + jnp.dot(xe_ref[...].astype(bf), m_ref[0],
                   preferred_element_type=jnp.float32)
         + jnp.dot(xs_ref[...].astype(bf), wc_ref[...],
                   preferred_element_type=jnp.float32)
         + b0)
    h = jnp.maximum(h, 0.0).astype(bf)
    h = jnp.maximum(jnp.dot(h, w1_ref[...], preferred_element_type=jnp.float32)
                    + b1_ref[...], 0.0).astype(bf)
    out_ref[...] = jnp.dot(h, w2_ref[...], preferred_element_type=jnp.float32) + b2_ref[...]


def _cycle_mlp(xc, xe, xs, wa, m2, wc, b02, w1, b1, w2, b2, blk, nblk5):
    r, d = xc.shape
    grid = r // blk
    row = lambda i: (i, 0)
    full = lambda i: (0, 0)
    size_sel = lambda i: (jnp.where(i < nblk5, 0, 1), 0, 0)
    return pl.pallas_call(
        functools.partial(_cycle_mlp_body, nblk5),
        grid=(grid,),
        in_specs=[
            pl.BlockSpec((blk, d), row),
            pl.BlockSpec((blk, d), row),
            pl.BlockSpec((blk, d), row),
            pl.BlockSpec(wa.shape, full),
            pl.BlockSpec((1,) + m2.shape[1:], size_sel),
            pl.BlockSpec(wc.shape, full),
            pl.BlockSpec(b02.shape, full),
            pl.BlockSpec(w1.shape, full),
            pl.BlockSpec(b1.shape, lambda i: (0,)),
            pl.BlockSpec(w2.shape, full),
            pl.BlockSpec(b2.shape, lambda i: (0,)),
        ],
        out_specs=pl.BlockSpec((blk, d), row),
        out_shape=jax.ShapeDtypeStruct((r, d), jnp.float32),
    )(xc, xe, xs, wa, m2, wc, b02, w1, b1, w2, b2)


def _edge_mlp_body(xe_ref, xc_ref, wa_ref, wb_ref, b0_ref, w1_ref, b1_ref, out_ref):
    bf = jnp.bfloat16
    h = (jnp.dot(xe_ref[...].astype(bf), wa_ref[...],
                 preferred_element_type=jnp.float32)
         + jnp.dot(xc_ref[...].astype(bf), wb_ref[...],
                   preferred_element_type=jnp.float32)
         + b0_ref[...])
    h = jnp.maximum(h, 0.0).astype(bf)
    out_ref[...] = jnp.dot(h, w1_ref[...], preferred_element_type=jnp.float32) + b1_ref[...]


def _edge_mlp(xe, xc, wa, wb, b0, w1, b1, blk):
    r, d = xe.shape
    grid = r // blk
    row = lambda i: (i, 0)
    full = lambda i: (0, 0)
    return pl.pallas_call(
        _edge_mlp_body,
        grid=(grid,),
        in_specs=[
            pl.BlockSpec((blk, d), row),
            pl.BlockSpec((blk, d), row),
            pl.BlockSpec(wa.shape, full),
            pl.BlockSpec(wb.shape, full),
            pl.BlockSpec(b0.shape, lambda i: (0,)),
            pl.BlockSpec(w1.shape, full),
            pl.BlockSpec(b1.shape, lambda i: (0,)),
        ],
        out_specs=pl.BlockSpec((blk, d), row),
        out_shape=jax.ShapeDtypeStruct((r, d), jnp.float32),
    )(xe, xc, wa, wb, b0, w1, b1)


_OB = 512
_REG = 520
_CH = 128
_CAPC = 32


def _extract(vref, pos):
    base8 = (pos // 8) * 8
    lane = pos - base8
    v = vref[pl.ds(base8, 16)]
    val = v[0]
    for j in range(1, 8):
        val = jnp.where(lane == j, v[j], val)
    return val


def _sc_seg_body(nb, ob, d, table_h, src_h, dst_h, offa_h, offb_h, zeros_h,
                 out_h, offa_v, offb_v, src_w, dst_w, gidx2, src_s, dst_s,
                 gidx_s, rows_v, zero_v, acc_sh,
                 gsem0, gsem1, ssem0, ssem1):
    cid = lax.axis_index("c")
    sid = lax.axis_index("s")
    w = cid * 16 + sid
    nz = (_REG + 127) // 128
    pltpu.sync_copy(offa_h, offa_v)
    pltpu.sync_copy(offb_h, offb_v)
    pltpu.sync_copy(zeros_h, zero_v)
    for j in range(nz - 1):
        pltpu.sync_copy(zero_v, acc_sh.at[pl.ds(sid * _REG + j * 128, 128)])
    pltpu.sync_copy(zero_v.at[pl.ds(0, _REG - (nz - 1) * 128)],
                    acc_sh.at[pl.ds(sid * _REG + (nz - 1) * 128,
                                    _REG - (nz - 1) * 128)])

    def lidx_of(dvec, base):
        local = dvec - base
        valid = (local >= 0) & (local < ob)
        return jnp.where(valid, local, ob) + sid * _REG

    def issue_gather(idx_ref, buf, sem):
        pltpu.async_copy(table_h.at[idx_ref], buf, sem)

    def wait_rows(buf, sem):
        pltpu.make_async_copy(table_h.at[src_s], buf, sem).wait()

    def issue_scatter(idx_ref, buf, sem):
        pltpu.async_copy(buf, acc_sh.at[idx_ref], sem, add=True)

    def wait_scatter(buf, sem):
        pltpu.make_async_copy(buf, acc_sh.at[gidx_s.at[0]], sem).wait()

    def per_buf(k, fn):
        @pl.when(k % 2 == 0)
        def _():
            fn(rows_v.at[0], gsem0, ssem0)

        @pl.when(k % 2 == 1)
        def _():
            fn(rows_v.at[1], gsem1, ssem1)

    def block_body(kk, carry):
        b = w + kk * 32
        start = _extract(offa_v, b)
        end = _extract(offb_v, b)
        start8 = (start // 8) * 8
        nch = (end - start8 + _CH - 1) // _CH
        kfast = jnp.minimum(nch, _CAPC)
        base = b * ob

        pltpu.sync_copy(src_h.at[pl.ds(start8, _CAPC * _CH)], src_w)
        pltpu.sync_copy(dst_h.at[pl.ds(start8, _CAPC * _CH)], dst_w)

        def pre_body(ck, c2):
            for j in range(_CH // 16):
                dvec = dst_w[pl.ds(ck * _CH + j * 16, 16)]
                gidx2[ck, pl.ds(j * 16, 16)] = lidx_of(dvec, base)
            return c2

        lax.fori_loop(0, kfast, pre_body, 0)

        def start_fast(ck):
            per_buf(ck, lambda buf, gs, ss:
                    issue_gather(src_w.at[pl.ds(ck * _CH, _CH)], buf, gs))

        @pl.when(kfast > 0)
        def _():
            start_fast(0)

        def chunk_body(ck, c2):
            @pl.when(ck + 1 < kfast)
            def _():
                @pl.when(ck >= 1)
                def _():
                    per_buf(ck + 1, lambda buf, gs, ss: wait_scatter(buf, ss))
                start_fast(ck + 1)

            def fin(buf, gs, ss):
                wait_rows(buf, gs)
                issue_scatter(gidx2.at[ck], buf, ss)

            per_buf(ck, fin)
            return c2

        lax.fori_loop(0, kfast, chunk_body, 0)

        @pl.when(kfast >= 2)
        def _():
            per_buf(kfast - 2, lambda buf, gs, ss: wait_scatter(buf, ss))

        @pl.when(kfast >= 1)
        def _():
            per_buf(kfast - 1, lambda buf, gs, ss: wait_scatter(buf, ss))

        def slow_body(ck, c2):
            g0 = start8 + ck * _CH
            pltpu.sync_copy(src_h.at[pl.ds(g0, _CH)], src_s)
            pltpu.sync_copy(dst_h.at[pl.ds(g0, _CH)], dst_s)
            for j in range(_CH // 16):
                dvec = dst_s[pl.ds(j * 16, 16)]
                gidx_s[0, pl.ds(j * 16, 16)] = lidx_of(dvec, base)
            pltpu.async_copy(table_h.at[src_s], rows_v.at[0], gsem0)
            wait_rows(rows_v.at[0], gsem0)
            pltpu.async_copy(rows_v.at[0], acc_sh.at[gidx_s.at[0]], ssem0,
                             add=True)
            wait_scatter(rows_v.at[0], ssem0)
            return c2

        lax.fori_loop(kfast, nch, slow_body, 0)

        pltpu.sync_copy(acc_sh.at[pl.ds(sid * _REG, ob)],
                        out_h.at[pl.ds(base, ob)])
        for j in range(ob // 128):
            pltpu.sync_copy(zero_v, acc_sh.at[pl.ds(sid * _REG + j * 128, 128)])
        return carry

    lax.fori_loop(0, (nb - w + 31) // 32, block_body, 0)


def _seg_sum_gather(table, src, dst, num_out):
    n = src.shape[0]
    d = table.shape[1]
    ob = _OB
    nb = (num_out + ob - 1) // ob
    npad = nb * ob
    off = jnp.searchsorted(dst, jnp.arange(nb + 1, dtype=jnp.int32) * ob
                           ).astype(jnp.int32)
    padnb = 16 * ((nb + 31) // 16)
    filler = jnp.full((padnb - nb,), n, dtype=jnp.int32)
    offa = jnp.concatenate([off[:nb], filler])
    offb = jnp.concatenate([off[1:nb + 1], filler])
    pad = _CAPC * _CH
    src_p = jnp.concatenate([src.astype(jnp.int32),
                             jnp.zeros((pad,), dtype=jnp.int32)])
    dst_p = jnp.concatenate([dst.astype(jnp.int32),
                             jnp.full((pad,), npad, dtype=jnp.int32)])
    zeros = jnp.zeros((128, d), dtype=jnp.float32)

    mesh = plsc.VectorSubcoreMesh(core_axis_name="c", subcore_axis_name="s")
    body = functools.partial(_sc_seg_body, nb, ob, d)
    f = pl.kernel(
        body,
        out_type=jax.ShapeDtypeStruct((npad, d), jnp.float32),
        mesh=mesh,
        scratch_types=[
            pltpu.VMEM((padnb,), jnp.int32),
            pltpu.VMEM((padnb,), jnp.int32),
            pltpu.VMEM((_CAPC * _CH,), jnp.int32),
            pltpu.VMEM((_CAPC * _CH,), jnp.int32),
            pltpu.VMEM((_CAPC, _CH), jnp.int32),
            pltpu.VMEM((_CH,), jnp.int32),
            pltpu.VMEM((_CH,), jnp.int32),
            pltpu.VMEM((1, _CH), jnp.int32),
            pltpu.VMEM((2, _CH, d), jnp.float32),
            pltpu.VMEM((128, d), jnp.float32),
            pltpu.VMEM_SHARED((16 * _REG, d), jnp.float32),
            pltpu.SemaphoreType.DMA,
            pltpu.SemaphoreType.DMA,
            pltpu.SemaphoreType.DMA,
            pltpu.SemaphoreType.DMA,
        ],
    )
    return f(table, src_p, dst_p, offa, offb, zeros)


def kernel(edge_rep, cycle_rep, e2c_src_5, e2c_dst_5, e2c_src_6, e2c_dst_6,
           c2c_src_5, c2c_dst_5, c2c_src_6, c2c_dst_6, c2e_src, c2e_dst,
           lin_w_5, lin_b_5, lin_w_6, lin_b_6,
           cmlp_w0, cmlp_b0, cmlp_w1, cmlp_b1, cmlp_w2, cmlp_b2,
           emlp_w0, emlp_b0, emlp_w1, emlp_b1):
    nE, d = edge_rep.shape
    r5 = c2c_src_5.shape[0]
    r6 = c2c_src_6.shape[0]

    w0a = cmlp_w0[:d]
    w0b = cmlp_w0[d:3 * d]
    w0c = cmlp_w0[3 * d:4 * d]
    w0d = cmlp_w0[4 * d:]
    m5 = lin_w_5 @ w0b + w0d
    m6 = lin_w_6 @ w0b + w0d
    b0_5 = cmlp_b0 + lin_b_5 @ w0b
    b0_6 = cmlp_b0 + lin_b_6 @ w0b

    e2c_src = jnp.concatenate([e2c_src_5, e2c_src_6]).astype(jnp.int32)
    e2c_dst = jnp.concatenate([e2c_dst_5, e2c_dst_6 + r5]).astype(jnp.int32)
    e2c = _seg_sum_gather(edge_rep, e2c_src, e2c_dst, r5 + r6)
    c2c_src = jnp.concatenate([c2c_src_5, c2c_src_6 + r5]).astype(jnp.int32)
    c2c_dst = jnp.concatenate([c2c_dst_5, c2c_dst_6 + r5]).astype(jnp.int32)
    s = _seg_sum_gather(e2c, c2c_src, c2c_dst, r5 + r6)

    blk = 1000
    bf = jnp.bfloat16
    cycle_out = _cycle_mlp(cycle_rep, e2c, s, w0a.astype(bf),
                           jnp.stack([m5, m6]).astype(bf), w0c.astype(bf),
                           jnp.stack([b0_5, b0_6]),
                           cmlp_w1.astype(bf), cmlp_b1, cmlp_w2.astype(bf),
                           cmlp_b2, blk=blk, nblk5=r5 // blk)

    c2e = _seg_sum_gather(cycle_out, c2e_src, c2e_dst, nE)

    e0a = emlp_w0[:d]
    e0b = emlp_w0[d:]
    edge_out = _edge_mlp(edge_rep, c2e, e0a.astype(bf), e0b.astype(bf),
                         emlp_b0, emlp_w1.astype(bf), emlp_b1, blk=1000)
    return (edge_out, cycle_out)


# speedup vs baseline: 1.2013x; 1.2013x over previous
import functools

import jax
import jax.numpy as jnp
from jax import lax
from jax.experimental import pallas as pl
from jax.experimental.pallas import tpu as pltpu
from jax.experimental.pallas import tpu_sc as plsc


def _cycle_mlp_body(nblk5, xc_ref, xe_ref, xs_ref, w0_ref, b0_ref, w1_ref,
                    b1_ref, w2_ref, b2_ref, out_ref):
    bf = jnp.bfloat16
    b0 = jnp.where(pl.program_id(0) < nblk5, b0_ref[0], b0_ref[1])
    x = jnp.concatenate([xc_ref[...], xe_ref[...], xs_ref[...]],
                        axis=1).astype(bf)
    h = jnp.dot(x, w0_ref[0], preferred_element_type=jnp.float32) + b0
    h = jnp.maximum(h, 0.0).astype(bf)
    h = jnp.maximum(jnp.dot(h, w1_ref[...], preferred_element_type=jnp.float32)
                    + b1_ref[...], 0.0).astype(bf)
    out_ref[...] = jnp.dot(h, w2_ref[...], preferred_element_type=jnp.float32) + b2_ref[...]


def _cycle_mlp(xc, xe, xs, w02, b02, w1, b1, w2, b2, blk, nblk5):
    r, d = xc.shape
    grid = r // blk
    row = lambda i: (i, 0)
    full = lambda i: (0, 0)
    size_sel = lambda i: (jnp.where(i < nblk5, 0, 1), 0, 0)
    return pl.pallas_call(
        functools.partial(_cycle_mlp_body, nblk5),
        grid=(grid,),
        in_specs=[
            pl.BlockSpec((blk, d), row),
            pl.BlockSpec((blk, d), row),
            pl.BlockSpec((blk, d), row),
            pl.BlockSpec((1,) + w02.shape[1:], size_sel),
            pl.BlockSpec(b02.shape, full),
            pl.BlockSpec(w1.shape, full),
            pl.BlockSpec(b1.shape, lambda i: (0,)),
            pl.BlockSpec(w2.shape, full),
            pl.BlockSpec(b2.shape, lambda i: (0,)),
        ],
        out_specs=pl.BlockSpec((blk, d), row),
        out_shape=jax.ShapeDtypeStruct((r, d), jnp.float32),
    )(xc, xe, xs, w02, b02, w1, b1, w2, b2)


def _edge_mlp_body(xe_ref, xc_ref, w0_ref, b0_ref, w1_ref, b1_ref, out_ref):
    bf = jnp.bfloat16
    x = jnp.concatenate([xe_ref[...], xc_ref[...]], axis=1).astype(bf)
    h = jnp.dot(x, w0_ref[...], preferred_element_type=jnp.float32) + b0_ref[...]
    h = jnp.maximum(h, 0.0).astype(bf)
    out_ref[...] = jnp.dot(h, w1_ref[...], preferred_element_type=jnp.float32) + b1_ref[...]


def _edge_mlp(xe, xc, w0, b0, w1, b1, blk):
    r, d = xe.shape
    grid = r // blk
    row = lambda i: (i, 0)
    full = lambda i: (0, 0)
    return pl.pallas_call(
        _edge_mlp_body,
        grid=(grid,),
        in_specs=[
            pl.BlockSpec((blk, d), row),
            pl.BlockSpec((blk, d), row),
            pl.BlockSpec(w0.shape, full),
            pl.BlockSpec(b0.shape, lambda i: (0,)),
            pl.BlockSpec(w1.shape, full),
            pl.BlockSpec(b1.shape, lambda i: (0,)),
        ],
        out_specs=pl.BlockSpec((blk, d), row),
        out_shape=jax.ShapeDtypeStruct((r, d), jnp.float32),
    )(xe, xc, w0, b0, w1, b1)


_OB = 512
_REG = 520
_CH = 128
_CAPC = 32


def _extract(vref, pos):
    base8 = (pos // 8) * 8
    lane = pos - base8
    v = vref[pl.ds(base8, 16)]
    val = v[0]
    for j in range(1, 8):
        val = jnp.where(lane == j, v[j], val)
    return val


def _sc_seg_body(nb, ob, d, table_h, src_h, dst_h, offa_h, offb_h, zeros_h,
                 out_h, offa_v, offb_v, src_w, dst_w, gidx2, src_s, dst_s,
                 gidx_s, rows_v, zero_v, acc_sh,
                 gsem0, gsem1, ssem0, ssem1):
    cid = lax.axis_index("c")
    sid = lax.axis_index("s")
    w = cid * 16 + sid
    nz = (_REG + 127) // 128
    pltpu.sync_copy(offa_h, offa_v)
    pltpu.sync_copy(offb_h, offb_v)
    pltpu.sync_copy(zeros_h, zero_v)
    for j in range(nz - 1):
        pltpu.sync_copy(zero_v, acc_sh.at[pl.ds(sid * _REG + j * 128, 128)])
    pltpu.sync_copy(zero_v.at[pl.ds(0, _REG - (nz - 1) * 128)],
                    acc_sh.at[pl.ds(sid * _REG + (nz - 1) * 128,
                                    _REG - (nz - 1) * 128)])

    def lidx_of(dvec, base):
        local = dvec - base
        valid = (local >= 0) & (local < ob)
        return jnp.where(valid, local, ob) + sid * _REG

    def issue_gather(idx_ref, buf, sem):
        pltpu.async_copy(table_h.at[idx_ref], buf, sem)

    def wait_rows(buf, sem):
        pltpu.make_async_copy(table_h.at[src_s], buf, sem).wait()

    def issue_scatter(idx_ref, buf, sem):
        pltpu.async_copy(buf, acc_sh.at[idx_ref], sem, add=True)

    def wait_scatter(buf, sem):
        pltpu.make_async_copy(buf, acc_sh.at[gidx_s.at[0]], sem).wait()

    def per_buf(k, fn):
        @pl.when(k % 2 == 0)
        def _():
            fn(rows_v.at[0], gsem0, ssem0)

        @pl.when(k % 2 == 1)
        def _():
            fn(rows_v.at[1], gsem1, ssem1)

    def block_body(kk, carry):
        b = w + kk * 32
        start = _extract(offa_v, b)
        end = _extract(offb_v, b)
        start8 = (start // 8) * 8
        nch = (end - start8 + _CH - 1) // _CH
        kfast = jnp.minimum(nch, _CAPC)
        base = b * ob

        pltpu.sync_copy(src_h.at[pl.ds(start8, _CAPC * _CH)], src_w)
        pltpu.sync_copy(dst_h.at[pl.ds(start8, _CAPC * _CH)], dst_w)

        def pre_body(ck, c2):
            for j in range(_CH // 16):
                dvec = dst_w[pl.ds(ck * _CH + j * 16, 16)]
                gidx2[ck, pl.ds(j * 16, 16)] = lidx_of(dvec, base)
            return c2

        lax.fori_loop(0, kfast, pre_body, 0)

        def start_fast(ck):
            per_buf(ck, lambda buf, gs, ss:
                    issue_gather(src_w.at[pl.ds(ck * _CH, _CH)], buf, gs))

        @pl.when(kfast > 0)
        def _():
            start_fast(0)

        def chunk_body(ck, c2):
            @pl.when(ck + 1 < kfast)
            def _():
                @pl.when(ck >= 1)
                def _():
                    per_buf(ck + 1, lambda buf, gs, ss: wait_scatter(buf, ss))
                start_fast(ck + 1)

            def fin(buf, gs, ss):
                wait_rows(buf, gs)
                issue_scatter(gidx2.at[ck], buf, ss)

            per_buf(ck, fin)
            return c2

        lax.fori_loop(0, kfast, chunk_body, 0)

        @pl.when(kfast >= 2)
        def _():
            per_buf(kfast - 2, lambda buf, gs, ss: wait_scatter(buf, ss))

        @pl.when(kfast >= 1)
        def _():
            per_buf(kfast - 1, lambda buf, gs, ss: wait_scatter(buf, ss))

        def slow_body(ck, c2):
            g0 = start8 + ck * _CH
            pltpu.sync_copy(src_h.at[pl.ds(g0, _CH)], src_s)
            pltpu.sync_copy(dst_h.at[pl.ds(g0, _CH)], dst_s)
            for j in range(_CH // 16):
                dvec = dst_s[pl.ds(j * 16, 16)]
                gidx_s[0, pl.ds(j * 16, 16)] = lidx_of(dvec, base)
            pltpu.async_copy(table_h.at[src_s], rows_v.at[0], gsem0)
            wait_rows(rows_v.at[0], gsem0)
            pltpu.async_copy(rows_v.at[0], acc_sh.at[gidx_s.at[0]], ssem0,
                             add=True)
            wait_scatter(rows_v.at[0], ssem0)
            return c2

        lax.fori_loop(kfast, nch, slow_body, 0)

        pltpu.sync_copy(acc_sh.at[pl.ds(sid * _REG, ob)],
                        out_h.at[pl.ds(base, ob)])
        for j in range(ob // 128):
            pltpu.sync_copy(zero_v, acc_sh.at[pl.ds(sid * _REG + j * 128, 128)])
        return carry

    lax.fori_loop(0, (nb - w + 31) // 32, block_body, 0)


def _seg_sum_gather(table, src, dst, num_out):
    n = src.shape[0]
    d = table.shape[1]
    ob = _OB
    nb = (num_out + ob - 1) // ob
    npad = nb * ob
    pad = _CAPC * _CH
    src_p = jnp.concatenate([src.astype(jnp.int32),
                             jnp.zeros((pad,), dtype=jnp.int32)])
    dst_p = jnp.concatenate([dst.astype(jnp.int32),
                             jnp.full((pad,), npad, dtype=jnp.int32)])
    cw = 1024
    nc = (n + cw - 1) // cw
    dmat = dst_p[:nc * cw].reshape(nc, cw)
    lead = dmat[:, 0]
    v = jnp.arange(nb + 1, dtype=jnp.int32) * ob
    j = jnp.maximum(
        jnp.searchsorted(lead, v, side='left', method='compare_all')
        .astype(jnp.int32) - 1, 0)
    rows = dmat[j]
    off = j * cw + jnp.sum(rows < v[:, None], axis=1, dtype=jnp.int32)
    padnb = 16 * ((nb + 31) // 16)
    filler = jnp.full((padnb - nb,), n, dtype=jnp.int32)
    offa = jnp.concatenate([off[:nb], filler])
    offb = jnp.concatenate([off[1:nb + 1], filler])
    zeros = jnp.zeros((128, d), dtype=jnp.float32)

    mesh = plsc.VectorSubcoreMesh(core_axis_name="c", subcore_axis_name="s")
    body = functools.partial(_sc_seg_body, nb, ob, d)
    f = pl.kernel(
        body,
        out_type=jax.ShapeDtypeStruct((npad, d), jnp.float32),
        mesh=mesh,
        scratch_types=[
            pltpu.VMEM((padnb,), jnp.int32),
            pltpu.VMEM((padnb,), jnp.int32),
            pltpu.VMEM((_CAPC * _CH,), jnp.int32),
            pltpu.VMEM((_CAPC * _CH,), jnp.int32),
            pltpu.VMEM((_CAPC, _CH), jnp.int32),
            pltpu.VMEM((_CH,), jnp.int32),
            pltpu.VMEM((_CH,), jnp.int32),
            pltpu.VMEM((1, _CH), jnp.int32),
            pltpu.VMEM((2, _CH, d), jnp.float32),
            pltpu.VMEM((128, d), jnp.float32),
            pltpu.VMEM_SHARED((16 * _REG, d), jnp.float32),
            pltpu.SemaphoreType.DMA,
            pltpu.SemaphoreType.DMA,
            pltpu.SemaphoreType.DMA,
            pltpu.SemaphoreType.DMA,
        ],
    )
    return f(table, src_p, dst_p, offa, offb, zeros)


def kernel(edge_rep, cycle_rep, e2c_src_5, e2c_dst_5, e2c_src_6, e2c_dst_6,
           c2c_src_5, c2c_dst_5, c2c_src_6, c2c_dst_6, c2e_src, c2e_dst,
           lin_w_5, lin_b_5, lin_w_6, lin_b_6,
           cmlp_w0, cmlp_b0, cmlp_w1, cmlp_b1, cmlp_w2, cmlp_b2,
           emlp_w0, emlp_b0, emlp_w1, emlp_b1):
    nE, d = edge_rep.shape
    r5 = c2c_src_5.shape[0]
    r6 = c2c_src_6.shape[0]

    w0a = cmlp_w0[:d]
    w0b = cmlp_w0[d:3 * d]
    w0c = cmlp_w0[3 * d:4 * d]
    w0d = cmlp_w0[4 * d:]
    m5 = lin_w_5 @ w0b + w0d
    m6 = lin_w_6 @ w0b + w0d
    b0_5 = cmlp_b0 + lin_b_5 @ w0b
    b0_6 = cmlp_b0 + lin_b_6 @ w0b

    e2c_src = jnp.concatenate([e2c_src_5, e2c_src_6]).astype(jnp.int32)
    e2c_dst = jnp.concatenate([e2c_dst_5, e2c_dst_6 + r5]).astype(jnp.int32)
    e2c = _seg_sum_gather(edge_rep, e2c_src, e2c_dst, r5 + r6)
    c2c_src = jnp.concatenate([c2c_src_5, c2c_src_6 + r5]).astype(jnp.int32)
    c2c_dst = jnp.concatenate([c2c_dst_5, c2c_dst_6 + r5]).astype(jnp.int32)
    s = _seg_sum_gather(e2c, c2c_src, c2c_dst, r5 + r6)

    blk = 2000
    bf = jnp.bfloat16
    w02 = jnp.stack([jnp.concatenate([w0a, m5, w0c], axis=0),
                     jnp.concatenate([w0a, m6, w0c], axis=0)]).astype(bf)
    cycle_out = _cycle_mlp(cycle_rep, e2c, s, w02,
                           jnp.stack([b0_5, b0_6]),
                           cmlp_w1.astype(bf), cmlp_b1, cmlp_w2.astype(bf),
                           cmlp_b2, blk=blk, nblk5=r5 // blk)

    c2e = _seg_sum_gather(cycle_out, c2e_src, c2e_dst, nE)

    edge_out = _edge_mlp(edge_rep, c2e, emlp_w0.astype(bf),
                         emlp_b0, emlp_w1.astype(bf), emlp_b1, blk=2000)
    return (edge_out, cycle_out)


# speedup vs baseline: 1.3007x; 1.0828x over previous
import functools

import jax
import jax.numpy as jnp
from jax import lax
from jax.experimental import pallas as pl
from jax.experimental.pallas import tpu as pltpu
from jax.experimental.pallas import tpu_sc as plsc


def _cycle_mlp_body(nblk5, xc_ref, xe_ref, xs_ref, w0_ref, b0_ref, w1_ref,
                    b1_ref, w2_ref, b2_ref, out_ref):
    bf = jnp.bfloat16
    b0 = jnp.where(pl.program_id(0) < nblk5, b0_ref[0], b0_ref[1])
    x = jnp.concatenate([xc_ref[...], xe_ref[...], xs_ref[...]],
                        axis=1).astype(bf)
    h = jnp.dot(x, w0_ref[0], preferred_element_type=jnp.float32) + b0
    h = jnp.maximum(h, 0.0).astype(bf)
    h = jnp.maximum(jnp.dot(h, w1_ref[...], preferred_element_type=jnp.float32)
                    + b1_ref[...], 0.0).astype(bf)
    out_ref[...] = jnp.dot(h, w2_ref[...], preferred_element_type=jnp.float32) + b2_ref[...]


def _cycle_mlp(xc, xe, xs, w02, b02, w1, b1, w2, b2, blk, nblk5):
    r, d = xc.shape
    grid = r // blk
    row = lambda i: (i, 0)
    full = lambda i: (0, 0)
    size_sel = lambda i: (jnp.where(i < nblk5, 0, 1), 0, 0)
    return pl.pallas_call(
        functools.partial(_cycle_mlp_body, nblk5),
        grid=(grid,),
        in_specs=[
            pl.BlockSpec((blk, d), row),
            pl.BlockSpec((blk, d), row),
            pl.BlockSpec((blk, d), row),
            pl.BlockSpec((1,) + w02.shape[1:], size_sel),
            pl.BlockSpec(b02.shape, full),
            pl.BlockSpec(w1.shape, full),
            pl.BlockSpec(b1.shape, lambda i: (0,)),
            pl.BlockSpec(w2.shape, full),
            pl.BlockSpec(b2.shape, lambda i: (0,)),
        ],
        out_specs=pl.BlockSpec((blk, d), row),
        out_shape=jax.ShapeDtypeStruct((r, d), jnp.float32),
    )(xc, xe, xs, w02, b02, w1, b1, w2, b2)


def _edge_mlp_body(xe_ref, xc_ref, w0_ref, b0_ref, w1_ref, b1_ref, out_ref):
    bf = jnp.bfloat16
    x = jnp.concatenate([xe_ref[...], xc_ref[...]], axis=1).astype(bf)
    h = jnp.dot(x, w0_ref[...], preferred_element_type=jnp.float32) + b0_ref[...]
    h = jnp.maximum(h, 0.0).astype(bf)
    out_ref[...] = jnp.dot(h, w1_ref[...], preferred_element_type=jnp.float32) + b1_ref[...]


def _edge_mlp(xe, xc, w0, b0, w1, b1, blk):
    r, d = xe.shape
    grid = r // blk
    row = lambda i: (i, 0)
    full = lambda i: (0, 0)
    return pl.pallas_call(
        _edge_mlp_body,
        grid=(grid,),
        in_specs=[
            pl.BlockSpec((blk, d), row),
            pl.BlockSpec((blk, d), row),
            pl.BlockSpec(w0.shape, full),
            pl.BlockSpec(b0.shape, lambda i: (0,)),
            pl.BlockSpec(w1.shape, full),
            pl.BlockSpec(b1.shape, lambda i: (0,)),
        ],
        out_specs=pl.BlockSpec((blk, d), row),
        out_shape=jax.ShapeDtypeStruct((r, d), jnp.float32),
    )(xe, xc, w0, b0, w1, b1)


_OB = 512
_REG = 520
_CH = 128
_CAPC = 32


def _extract(vref, pos):
    base8 = (pos // 8) * 8
    lane = pos - base8
    v = vref[pl.ds(base8, 16)]
    val = v[0]
    for j in range(1, 8):
        val = jnp.where(lane == j, v[j], val)
    return val


def _sc_seg_body(nb, ob, d, capc, table_h, src_h, dst_h, offa_h, offb_h, zeros_h,
                 out_h, offa_v, offb_v, src_w, dst_w, gidx2, src_s, dst_s,
                 gidx_s, rows_v, zero_v, acc_sh,
                 gsem0, gsem1, ssem0, ssem1):
    cid = lax.axis_index("c")
    sid = lax.axis_index("s")
    w = cid * 16 + sid
    nz = (_REG + 127) // 128
    pltpu.sync_copy(offa_h, offa_v)
    pltpu.sync_copy(offb_h, offb_v)
    pltpu.sync_copy(zeros_h, zero_v)
    for j in range(nz - 1):
        pltpu.sync_copy(zero_v, acc_sh.at[pl.ds(sid * _REG + j * 128, 128)])
    pltpu.sync_copy(zero_v.at[pl.ds(0, _REG - (nz - 1) * 128)],
                    acc_sh.at[pl.ds(sid * _REG + (nz - 1) * 128,
                                    _REG - (nz - 1) * 128)])

    def lidx_of(dvec, base):
        local = dvec - base
        valid = (local >= 0) & (local < ob)
        return jnp.where(valid, local, ob) + sid * _REG

    def issue_gather(idx_ref, buf, sem):
        pltpu.async_copy(table_h.at[idx_ref], buf, sem)

    def wait_rows(buf, sem):
        pltpu.make_async_copy(table_h.at[src_s], buf, sem).wait()

    def issue_scatter(idx_ref, buf, sem):
        pltpu.async_copy(buf, acc_sh.at[idx_ref], sem, add=True)

    def wait_scatter(buf, sem):
        pltpu.make_async_copy(buf, acc_sh.at[gidx_s.at[0]], sem).wait()

    def per_buf(k, fn):
        @pl.when(k % 2 == 0)
        def _():
            fn(rows_v.at[0], gsem0, ssem0)

        @pl.when(k % 2 == 1)
        def _():
            fn(rows_v.at[1], gsem1, ssem1)

    def block_body(kk, carry):
        b = w + kk * 32
        start = _extract(offa_v, b)
        end = _extract(offb_v, b)
        start8 = (start // 8) * 8
        nch = (end - start8 + _CH - 1) // _CH
        kfast = jnp.minimum(nch, capc)
        base = b * ob

        pltpu.sync_copy(src_h.at[pl.ds(start8, capc * _CH)], src_w)
        pltpu.sync_copy(dst_h.at[pl.ds(start8, capc * _CH)], dst_w)

        def pre_body(ck, c2):
            for j in range(_CH // 16):
                dvec = dst_w[pl.ds(ck * _CH + j * 16, 16)]
                gidx2[ck, pl.ds(j * 16, 16)] = lidx_of(dvec, base)
            return c2

        lax.fori_loop(0, kfast, pre_body, 0)

        def start_fast(ck):
            per_buf(ck, lambda buf, gs, ss:
                    issue_gather(src_w.at[pl.ds(ck * _CH, _CH)], buf, gs))

        @pl.when(kfast > 0)
        def _():
            start_fast(0)

        def chunk_body(ck, c2):
            @pl.when(ck + 1 < kfast)
            def _():
                @pl.when(ck >= 1)
                def _():
                    per_buf(ck + 1, lambda buf, gs, ss: wait_scatter(buf, ss))
                start_fast(ck + 1)

            def fin(buf, gs, ss):
                wait_rows(buf, gs)
                issue_scatter(gidx2.at[ck], buf, ss)

            per_buf(ck, fin)
            return c2

        lax.fori_loop(0, kfast, chunk_body, 0)

        @pl.when(kfast >= 2)
        def _():
            per_buf(kfast - 2, lambda buf, gs, ss: wait_scatter(buf, ss))

        @pl.when(kfast >= 1)
        def _():
            per_buf(kfast - 1, lambda buf, gs, ss: wait_scatter(buf, ss))

        def slow_body(ck, c2):
            g0 = start8 + ck * _CH
            pltpu.sync_copy(src_h.at[pl.ds(g0, _CH)], src_s)
            pltpu.sync_copy(dst_h.at[pl.ds(g0, _CH)], dst_s)
            for j in range(_CH // 16):
                dvec = dst_s[pl.ds(j * 16, 16)]
                gidx_s[0, pl.ds(j * 16, 16)] = lidx_of(dvec, base)
            pltpu.async_copy(table_h.at[src_s], rows_v.at[0], gsem0)
            wait_rows(rows_v.at[0], gsem0)
            pltpu.async_copy(rows_v.at[0], acc_sh.at[gidx_s.at[0]], ssem0,
                             add=True)
            wait_scatter(rows_v.at[0], ssem0)
            return c2

        lax.fori_loop(kfast, nch, slow_body, 0)

        pltpu.sync_copy(acc_sh.at[pl.ds(sid * _REG, ob)],
                        out_h.at[pl.ds(base, ob)])
        for j in range(ob // 128):
            pltpu.sync_copy(zero_v, acc_sh.at[pl.ds(sid * _REG + j * 128, 128)])
        return carry

    lax.fori_loop(0, (nb - w + 31) // 32, block_body, 0)


def _seg_sum_gather(table, src, dst, num_out):
    n = src.shape[0]
    d = table.shape[1]
    ob = _OB
    nb = (num_out + ob - 1) // ob
    npad = nb * ob
    capc = min(_CAPC, 3 * max(1, -(-n // (nb * _CH))))
    pad = capc * _CH
    src_p = jnp.concatenate([src.astype(jnp.int32),
                             jnp.zeros((pad,), dtype=jnp.int32)])
    dst_p = jnp.concatenate([dst.astype(jnp.int32),
                             jnp.full((pad,), npad, dtype=jnp.int32)])
    cw = 1024
    nc = (n + cw - 1) // cw
    dmat = dst_p[:nc * cw].reshape(nc, cw)
    lead = dmat[:, 0]
    v = jnp.arange(nb + 1, dtype=jnp.int32) * ob
    j = jnp.maximum(
        jnp.searchsorted(lead, v, side='left', method='compare_all')
        .astype(jnp.int32) - 1, 0)
    rows = dmat[j]
    off = j * cw + jnp.sum(rows < v[:, None], axis=1, dtype=jnp.int32)
    padnb = 16 * ((nb + 31) // 16)
    filler = jnp.full((padnb - nb,), n, dtype=jnp.int32)
    offa = jnp.concatenate([off[:nb], filler])
    offb = jnp.concatenate([off[1:nb + 1], filler])
    zeros = jnp.zeros((128, d), dtype=jnp.float32)

    mesh = plsc.VectorSubcoreMesh(core_axis_name="c", subcore_axis_name="s")
    body = functools.partial(_sc_seg_body, nb, ob, d, capc)
    f = pl.kernel(
        body,
        out_type=jax.ShapeDtypeStruct((npad, d), jnp.float32),
        mesh=mesh,
        scratch_types=[
            pltpu.VMEM((padnb,), jnp.int32),
            pltpu.VMEM((padnb,), jnp.int32),
            pltpu.VMEM((capc * _CH,), jnp.int32),
            pltpu.VMEM((capc * _CH,), jnp.int32),
            pltpu.VMEM((capc, _CH), jnp.int32),
            pltpu.VMEM((_CH,), jnp.int32),
            pltpu.VMEM((_CH,), jnp.int32),
            pltpu.VMEM((1, _CH), jnp.int32),
            pltpu.VMEM((2, _CH, d), jnp.float32),
            pltpu.VMEM((128, d), jnp.float32),
            pltpu.VMEM_SHARED((16 * _REG, d), jnp.float32),
            pltpu.SemaphoreType.DMA,
            pltpu.SemaphoreType.DMA,
            pltpu.SemaphoreType.DMA,
            pltpu.SemaphoreType.DMA,
        ],
    )
    return f(table, src_p, dst_p, offa, offb, zeros)


def kernel(edge_rep, cycle_rep, e2c_src_5, e2c_dst_5, e2c_src_6, e2c_dst_6,
           c2c_src_5, c2c_dst_5, c2c_src_6, c2c_dst_6, c2e_src, c2e_dst,
           lin_w_5, lin_b_5, lin_w_6, lin_b_6,
           cmlp_w0, cmlp_b0, cmlp_w1, cmlp_b1, cmlp_w2, cmlp_b2,
           emlp_w0, emlp_b0, emlp_w1, emlp_b1):
    nE, d = edge_rep.shape
    r5 = c2c_src_5.shape[0]
    r6 = c2c_src_6.shape[0]

    w0a = cmlp_w0[:d]
    w0b = cmlp_w0[d:3 * d]
    w0c = cmlp_w0[3 * d:4 * d]
    w0d = cmlp_w0[4 * d:]
    m5 = lin_w_5 @ w0b + w0d
    m6 = lin_w_6 @ w0b + w0d
    b0_5 = cmlp_b0 + lin_b_5 @ w0b
    b0_6 = cmlp_b0 + lin_b_6 @ w0b

    e2c_src = jnp.concatenate([e2c_src_5, e2c_src_6]).astype(jnp.int32)
    e2c_dst = jnp.concatenate([e2c_dst_5, e2c_dst_6 + r5]).astype(jnp.int32)
    e2c = _seg_sum_gather(edge_rep, e2c_src, e2c_dst, r5 + r6)
    c2c_src = jnp.concatenate([c2c_src_5, c2c_src_6 + r5]).astype(jnp.int32)
    c2c_dst = jnp.concatenate([c2c_dst_5, c2c_dst_6 + r5]).astype(jnp.int32)
    s = _seg_sum_gather(e2c, c2c_src, c2c_dst, r5 + r6)

    blk = 4000
    bf = jnp.bfloat16
    w02 = jnp.stack([jnp.concatenate([w0a, m5, w0c], axis=0),
                     jnp.concatenate([w0a, m6, w0c], axis=0)]).astype(bf)
    cycle_out = _cycle_mlp(cycle_rep, e2c, s, w02,
                           jnp.stack([b0_5, b0_6]),
                           cmlp_w1.astype(bf), cmlp_b1, cmlp_w2.astype(bf),
                           cmlp_b2, blk=blk, nblk5=r5 // blk)

    c2e = _seg_sum_gather(cycle_out, c2e_src, c2e_dst, nE)

    edge_out = _edge_mlp(edge_rep, c2e, emlp_w0.astype(bf),
                         emlp_b0, emlp_w1.astype(bf), emlp_b1, blk=4000)
    return (edge_out, cycle_out)


# speedup vs baseline: 1.3901x; 1.0687x over previous
import functools

import jax
import jax.numpy as jnp
from jax import lax
from jax.experimental import pallas as pl
from jax.experimental.pallas import tpu as pltpu
from jax.experimental.pallas import tpu_sc as plsc


def _cycle_mlp_body(nblk5, xc_ref, xe_ref, xs_ref, w0_ref, b0_ref, w1_ref,
                    b1_ref, w2_ref, b2_ref, out_ref):
    bf = jnp.bfloat16
    b0 = jnp.where(pl.program_id(0) < nblk5, b0_ref[0], b0_ref[1])
    x = jnp.concatenate([xc_ref[...], xe_ref[...], xs_ref[...]],
                        axis=1).astype(bf)
    h = jnp.dot(x, w0_ref[0], preferred_element_type=jnp.float32) + b0
    h = jnp.maximum(h, 0.0).astype(bf)
    h = jnp.maximum(jnp.dot(h, w1_ref[...], preferred_element_type=jnp.float32)
                    + b1_ref[...], 0.0).astype(bf)
    out_ref[...] = jnp.dot(h, w2_ref[...], preferred_element_type=jnp.float32) + b2_ref[...]


def _cycle_mlp(xc, xe, xs, w02, b02, w1, b1, w2, b2, blk, nblk5):
    r, d = xc.shape
    grid = r // blk
    row = lambda i: (i, 0)
    full = lambda i: (0, 0)
    size_sel = lambda i: (jnp.where(i < nblk5, 0, 1), 0, 0)
    return pl.pallas_call(
        functools.partial(_cycle_mlp_body, nblk5),
        grid=(grid,),
        in_specs=[
            pl.BlockSpec((blk, d), row),
            pl.BlockSpec((blk, d), row),
            pl.BlockSpec((blk, d), row),
            pl.BlockSpec((1,) + w02.shape[1:], size_sel),
            pl.BlockSpec(b02.shape, full),
            pl.BlockSpec(w1.shape, full),
            pl.BlockSpec(b1.shape, lambda i: (0,)),
            pl.BlockSpec(w2.shape, full),
            pl.BlockSpec(b2.shape, lambda i: (0,)),
        ],
        out_specs=pl.BlockSpec((blk, d), row),
        out_shape=jax.ShapeDtypeStruct((r, d), jnp.float32),
    )(xc, xe, xs, w02, b02, w1, b1, w2, b2)


def _edge_mlp_body(xe_ref, xc_ref, w0_ref, b0_ref, w1_ref, b1_ref, out_ref):
    bf = jnp.bfloat16
    x = jnp.concatenate([xe_ref[...], xc_ref[...]], axis=1).astype(bf)
    h = jnp.dot(x, w0_ref[...], preferred_element_type=jnp.float32) + b0_ref[...]
    h = jnp.maximum(h, 0.0).astype(bf)
    out_ref[...] = jnp.dot(h, w1_ref[...], preferred_element_type=jnp.float32) + b1_ref[...]


def _edge_mlp(xe, xc, w0, b0, w1, b1, blk):
    r, d = xe.shape
    grid = r // blk
    row = lambda i: (i, 0)
    full = lambda i: (0, 0)
    return pl.pallas_call(
        _edge_mlp_body,
        grid=(grid,),
        in_specs=[
            pl.BlockSpec((blk, d), row),
            pl.BlockSpec((blk, d), row),
            pl.BlockSpec(w0.shape, full),
            pl.BlockSpec(b0.shape, lambda i: (0,)),
            pl.BlockSpec(w1.shape, full),
            pl.BlockSpec(b1.shape, lambda i: (0,)),
        ],
        out_specs=pl.BlockSpec((blk, d), row),
        out_shape=jax.ShapeDtypeStruct((r, d), jnp.float32),
    )(xe, xc, w0, b0, w1, b1)


_OB = 512
_REG = 520
_CH = 128
_CAPC = 32


def _extract(vref, pos):
    base8 = (pos // 8) * 8
    lane = pos - base8
    v = vref[pl.ds(base8, 16)]
    val = v[0]
    for j in range(1, 8):
        val = jnp.where(lane == j, v[j], val)
    return val


def _sc_seg_body(nb, ob, d, capc, table_h, src_h, dst_h, offa_h, offb_h, zeros_h,
                 out_h, offa_v, offb_v, src_w0, src_w1, dst_w0, dst_w1,
                 gidx2, src_s, dst_s,
                 gidx_s, rows_v, zero_v, acc_sh,
                 gsem0, gsem1, ssem0, ssem1, stsem, osem):
    src_ws = (src_w0, src_w1)
    dst_ws = (dst_w0, dst_w1)
    cid = lax.axis_index("c")
    sid = lax.axis_index("s")
    w = cid * 16 + sid
    nblk = (nb - w + 31) // 32
    nz = (_REG + 127) // 128
    pltpu.sync_copy(offa_h, offa_v)
    pltpu.sync_copy(offb_h, offb_v)
    pltpu.sync_copy(zeros_h, zero_v)
    for j in range(nz - 1):
        pltpu.sync_copy(zero_v, acc_sh.at[pl.ds(sid * _REG + j * 128, 128)])
    pltpu.sync_copy(zero_v.at[pl.ds(0, _REG - (nz - 1) * 128)],
                    acc_sh.at[pl.ds(sid * _REG + (nz - 1) * 128,
                                    _REG - (nz - 1) * 128)])

    def lidx_of(dvec, base):
        local = dvec - base
        valid = (local >= 0) & (local < ob)
        return jnp.where(valid, local, ob) + sid * _REG

    def issue_gather(idx_ref, buf, sem):
        pltpu.async_copy(table_h.at[idx_ref], buf, sem)

    def wait_rows(buf, sem):
        pltpu.make_async_copy(table_h.at[src_s], buf, sem).wait()

    def issue_scatter(idx_ref, buf, sem):
        pltpu.async_copy(buf, acc_sh.at[idx_ref], sem, add=True)

    def wait_scatter(buf, sem):
        pltpu.make_async_copy(buf, acc_sh.at[gidx_s.at[0]], sem).wait()

    def per_buf(k, fn):
        @pl.when(k % 2 == 0)
        def _():
            fn(rows_v.at[0], gsem0, ssem0)

        @pl.when(k % 2 == 1)
        def _():
            fn(rows_v.at[1], gsem1, ssem1)

    def stage_for(kk_next):
        b2 = w + kk_next * 32
        st8 = (_extract(offa_v, b2) // 8) * 8

        def issue(slot):
            pltpu.async_copy(src_h.at[pl.ds(st8, capc * _CH)],
                             src_ws[slot], stsem)
            pltpu.async_copy(dst_h.at[pl.ds(st8, capc * _CH)],
                             dst_ws[slot], stsem)

        @pl.when(kk_next % 2 == 0)
        def _():
            issue(0)

        @pl.when(kk_next % 2 == 1)
        def _():
            issue(1)

    def wait_stage():
        for _ in range(2):
            pltpu.make_async_copy(src_h.at[pl.ds(0, capc * _CH)],
                                  src_w0, stsem).wait()

    def wait_ocopy():
        pltpu.make_async_copy(acc_sh.at[pl.ds(sid * _REG, ob)],
                              out_h.at[pl.ds(0, ob)], osem).wait()

    @pl.when(nblk > 0)
    def _():
        stage_for(0)

    def block_body(kk, carry):
        b = w + kk * 32
        start = _extract(offa_v, b)
        end = _extract(offb_v, b)
        start8 = (start // 8) * 8
        nch = (end - start8 + _CH - 1) // _CH
        kfast = jnp.minimum(nch, capc)
        base = b * ob

        wait_stage()

        @pl.when(kk + 1 < nblk)
        def _():
            stage_for(kk + 1)

        @pl.when(kk > 0)
        def _():
            wait_ocopy()
            for j in range(ob // 128):
                pltpu.sync_copy(zero_v,
                                acc_sh.at[pl.ds(sid * _REG + j * 128, 128)])

        def process(slot):
            def pre_body(ck, c2):
                for j in range(_CH // 16):
                    dvec = dst_ws[slot][pl.ds(ck * _CH + j * 16, 16)]
                    gidx2[ck, pl.ds(j * 16, 16)] = lidx_of(dvec, base)
                return c2

            lax.fori_loop(0, kfast, pre_body, 0)

            def start_fast(ck):
                per_buf(ck, lambda buf, gs, ss:
                        issue_gather(src_ws[slot].at[pl.ds(ck * _CH, _CH)],
                                     buf, gs))

            @pl.when(kfast > 0)
            def _():
                start_fast(0)

            def chunk_body(ck, c2):
                @pl.when(ck + 1 < kfast)
                def _():
                    @pl.when(ck >= 1)
                    def _():
                        per_buf(ck + 1,
                                lambda buf, gs, ss: wait_scatter(buf, ss))
                    start_fast(ck + 1)

                def fin(buf, gs, ss):
                    wait_rows(buf, gs)
                    issue_scatter(gidx2.at[ck], buf, ss)

                per_buf(ck, fin)
                return c2

            lax.fori_loop(0, kfast, chunk_body, 0)

            @pl.when(kfast >= 2)
            def _():
                per_buf(kfast - 2, lambda buf, gs, ss: wait_scatter(buf, ss))

            @pl.when(kfast >= 1)
            def _():
                per_buf(kfast - 1, lambda buf, gs, ss: wait_scatter(buf, ss))

        @pl.when(kk % 2 == 0)
        def _():
            process(0)

        @pl.when(kk % 2 == 1)
        def _():
            process(1)

        def slow_body(ck, c2):
            g0 = start8 + ck * _CH
            pltpu.sync_copy(src_h.at[pl.ds(g0, _CH)], src_s)
            pltpu.sync_copy(dst_h.at[pl.ds(g0, _CH)], dst_s)
            for j in range(_CH // 16):
                dvec = dst_s[pl.ds(j * 16, 16)]
                gidx_s[0, pl.ds(j * 16, 16)] = lidx_of(dvec, base)
            pltpu.async_copy(table_h.at[src_s], rows_v.at[0], gsem0)
            wait_rows(rows_v.at[0], gsem0)
            pltpu.async_copy(rows_v.at[0], acc_sh.at[gidx_s.at[0]], ssem0,
                             add=True)
            wait_scatter(rows_v.at[0], ssem0)
            return c2

        lax.fori_loop(kfast, nch, slow_body, 0)

        pltpu.async_copy(acc_sh.at[pl.ds(sid * _REG, ob)],
                         out_h.at[pl.ds(base, ob)], osem)
        return carry

    lax.fori_loop(0, nblk, block_body, 0)

    @pl.when(nblk > 0)
    def _():
        wait_ocopy()


def _seg_sum_gather(table, src, dst, num_out):
    n = src.shape[0]
    d = table.shape[1]
    ob = _OB
    nb = (num_out + ob - 1) // ob
    npad = nb * ob
    capc = min(_CAPC, 2 * max(1, -(-n // (nb * _CH))))
    pad = capc * _CH
    src_p = jnp.concatenate([src.astype(jnp.int32),
                             jnp.zeros((pad,), dtype=jnp.int32)])
    dst_p = jnp.concatenate([dst.astype(jnp.int32),
                             jnp.full((pad,), npad, dtype=jnp.int32)])
    cw = 1024
    nc = (n + cw - 1) // cw
    dmat = dst_p[:nc * cw].reshape(nc, cw)
    lead = dmat[:, 0]
    v = jnp.arange(nb + 1, dtype=jnp.int32) * ob
    j = jnp.maximum(
        jnp.searchsorted(lead, v, side='left', method='compare_all')
        .astype(jnp.int32) - 1, 0)
    rows = dmat[j]
    off = j * cw + jnp.sum(rows < v[:, None], axis=1, dtype=jnp.int32)
    padnb = 16 * ((nb + 31) // 16)
    filler = jnp.full((padnb - nb,), n, dtype=jnp.int32)
    offa = jnp.concatenate([off[:nb], filler])
    offb = jnp.concatenate([off[1:nb + 1], filler])
    zeros = jnp.zeros((128, d), dtype=jnp.float32)

    mesh = plsc.VectorSubcoreMesh(core_axis_name="c", subcore_axis_name="s")
    body = functools.partial(_sc_seg_body, nb, ob, d, capc)
    f = pl.kernel(
        body,
        out_type=jax.ShapeDtypeStruct((npad, d), jnp.float32),
        mesh=mesh,
        scratch_types=[
            pltpu.VMEM((padnb,), jnp.int32),
            pltpu.VMEM((padnb,), jnp.int32),
            pltpu.VMEM((capc * _CH,), jnp.int32),
            pltpu.VMEM((capc * _CH,), jnp.int32),
            pltpu.VMEM((capc * _CH,), jnp.int32),
            pltpu.VMEM((capc * _CH,), jnp.int32),
            pltpu.VMEM((capc, _CH), jnp.int32),
            pltpu.VMEM((_CH,), jnp.int32),
            pltpu.VMEM((_CH,), jnp.int32),
            pltpu.VMEM((1, _CH), jnp.int32),
            pltpu.VMEM((2, _CH, d), jnp.float32),
            pltpu.VMEM((128, d), jnp.float32),
            pltpu.VMEM_SHARED((16 * _REG, d), jnp.float32),
            pltpu.SemaphoreType.DMA,
            pltpu.SemaphoreType.DMA,
            pltpu.SemaphoreType.DMA,
            pltpu.SemaphoreType.DMA,
            pltpu.SemaphoreType.DMA,
            pltpu.SemaphoreType.DMA,
        ],
    )
    return f(table, src_p, dst_p, offa, offb, zeros)


def kernel(edge_rep, cycle_rep, e2c_src_5, e2c_dst_5, e2c_src_6, e2c_dst_6,
           c2c_src_5, c2c_dst_5, c2c_src_6, c2c_dst_6, c2e_src, c2e_dst,
           lin_w_5, lin_b_5, lin_w_6, lin_b_6,
           cmlp_w0, cmlp_b0, cmlp_w1, cmlp_b1, cmlp_w2, cmlp_b2,
           emlp_w0, emlp_b0, emlp_w1, emlp_b1):
    nE, d = edge_rep.shape
    r5 = c2c_src_5.shape[0]
    r6 = c2c_src_6.shape[0]

    w0a = cmlp_w0[:d]
    w0b = cmlp_w0[d:3 * d]
    w0c = cmlp_w0[3 * d:4 * d]
    w0d = cmlp_w0[4 * d:]
    m5 = lin_w_5 @ w0b + w0d
    m6 = lin_w_6 @ w0b + w0d
    b0_5 = cmlp_b0 + lin_b_5 @ w0b
    b0_6 = cmlp_b0 + lin_b_6 @ w0b

    e2c_src = jnp.concatenate([e2c_src_5, e2c_src_6]).astype(jnp.int32)
    e2c_dst = jnp.concatenate([e2c_dst_5, e2c_dst_6 + r5]).astype(jnp.int32)
    e2c = _seg_sum_gather(edge_rep, e2c_src, e2c_dst, r5 + r6)
    c2c_src = jnp.concatenate([c2c_src_5, c2c_src_6 + r5]).astype(jnp.int32)
    c2c_dst = jnp.concatenate([c2c_dst_5, c2c_dst_6 + r5]).astype(jnp.int32)
    s = _seg_sum_gather(e2c, c2c_src, c2c_dst, r5 + r6)

    blk = 4000
    bf = jnp.bfloat16
    w02 = jnp.stack([jnp.concatenate([w0a, m5, w0c], axis=0),
                     jnp.concatenate([w0a, m6, w0c], axis=0)]).astype(bf)
    cycle_out = _cycle_mlp(cycle_rep, e2c, s, w02,
                           jnp.stack([b0_5, b0_6]),
                           cmlp_w1.astype(bf), cmlp_b1, cmlp_w2.astype(bf),
                           cmlp_b2, blk=blk, nblk5=r5 // blk)

    c2e = _seg_sum_gather(cycle_out, c2e_src, c2e_dst, nE)

    edge_out = _edge_mlp(edge_rep, c2e, emlp_w0.astype(bf),
                         emlp_b0, emlp_w1.astype(bf), emlp_b1, blk=4000)
    return (edge_out, cycle_out)


# speedup vs baseline: 1.4269x; 1.0264x over previous
import functools

import jax
import jax.numpy as jnp
from jax import lax
from jax.experimental import pallas as pl
from jax.experimental.pallas import tpu as pltpu
from jax.experimental.pallas import tpu_sc as plsc


def _cycle_mlp_body(nblk5, xc_ref, xe_ref, xs_ref, w0_ref, b0_ref, w1_ref,
                    b1_ref, w2_ref, b2_ref, out_ref):
    bf = jnp.bfloat16
    b0 = jnp.where(pl.program_id(0) < nblk5, b0_ref[0], b0_ref[1])
    x = jnp.concatenate([xc_ref[...], xe_ref[...], xs_ref[...]],
                        axis=1).astype(bf)
    h = jnp.dot(x, w0_ref[0], preferred_element_type=jnp.float32) + b0
    h = jnp.maximum(h, 0.0).astype(bf)
    h = jnp.maximum(jnp.dot(h, w1_ref[...], preferred_element_type=jnp.float32)
                    + b1_ref[...], 0.0).astype(bf)
    out_ref[...] = jnp.dot(h, w2_ref[...], preferred_element_type=jnp.float32) + b2_ref[...]


def _cycle_mlp(xc, xe, xs, w02, b02, w1, b1, w2, b2, blk, nblk5):
    r, d = xc.shape
    grid = r // blk
    row = lambda i: (i, 0)
    full = lambda i: (0, 0)
    size_sel = lambda i: (jnp.where(i < nblk5, 0, 1), 0, 0)
    return pl.pallas_call(
        functools.partial(_cycle_mlp_body, nblk5),
        grid=(grid,),
        in_specs=[
            pl.BlockSpec((blk, d), row),
            pl.BlockSpec((blk, d), row),
            pl.BlockSpec((blk, d), row),
            pl.BlockSpec((1,) + w02.shape[1:], size_sel),
            pl.BlockSpec(b02.shape, full),
            pl.BlockSpec(w1.shape, full),
            pl.BlockSpec(b1.shape, lambda i: (0,)),
            pl.BlockSpec(w2.shape, full),
            pl.BlockSpec(b2.shape, lambda i: (0,)),
        ],
        out_specs=pl.BlockSpec((blk, d), row),
        out_shape=jax.ShapeDtypeStruct((r, d), jnp.float32),
    )(xc, xe, xs, w02, b02, w1, b1, w2, b2)


def _edge_mlp_body(xe_ref, xc_ref, w0_ref, b0_ref, w1_ref, b1_ref, out_ref):
    bf = jnp.bfloat16
    x = jnp.concatenate([xe_ref[...], xc_ref[...]], axis=1).astype(bf)
    h = jnp.dot(x, w0_ref[...], preferred_element_type=jnp.float32) + b0_ref[...]
    h = jnp.maximum(h, 0.0).astype(bf)
    out_ref[...] = jnp.dot(h, w1_ref[...], preferred_element_type=jnp.float32) + b1_ref[...]


def _edge_mlp(xe, xc, w0, b0, w1, b1, blk):
    r, d = xe.shape
    grid = r // blk
    row = lambda i: (i, 0)
    full = lambda i: (0, 0)
    return pl.pallas_call(
        _edge_mlp_body,
        grid=(grid,),
        in_specs=[
            pl.BlockSpec((blk, d), row),
            pl.BlockSpec((blk, d), row),
            pl.BlockSpec(w0.shape, full),
            pl.BlockSpec(b0.shape, lambda i: (0,)),
            pl.BlockSpec(w1.shape, full),
            pl.BlockSpec(b1.shape, lambda i: (0,)),
        ],
        out_specs=pl.BlockSpec((blk, d), row),
        out_shape=jax.ShapeDtypeStruct((r, d), jnp.float32),
    )(xe, xc, w0, b0, w1, b1)


_OB = 512
_REG = 520
_CH = 128
_CAPC = 32


def _extract(vref, pos):
    base8 = (pos // 8) * 8
    lane = pos - base8
    v = vref[pl.ds(base8, 16)]
    val = v[0]
    for j in range(1, 8):
        val = jnp.where(lane == j, v[j], val)
    return val


def _sc_seg_body(nb, ob, d, capc, table_h, src_h, dst_h, offa_h, offb_h, zeros_h,
                 out_h, offa_v, offb_v, src_w0, src_w1, dst_w0, dst_w1,
                 gidx2, src_s, dst_s,
                 gidx_s, rows_v, zero_v, acc_sh,
                 gsem0, gsem1, ssem0, ssem1, stsem, osem):
    src_ws = (src_w0, src_w1)
    dst_ws = (dst_w0, dst_w1)
    cid = lax.axis_index("c")
    sid = lax.axis_index("s")
    w = cid * 16 + sid
    nblk = (nb - w + 31) // 32
    nz = (_REG + 127) // 128
    pltpu.sync_copy(offa_h, offa_v)
    pltpu.sync_copy(offb_h, offb_v)
    pltpu.sync_copy(zeros_h, zero_v)
    for j in range(nz - 1):
        pltpu.sync_copy(zero_v, acc_sh.at[pl.ds(sid * _REG + j * 128, 128)])
    pltpu.sync_copy(zero_v.at[pl.ds(0, _REG - (nz - 1) * 128)],
                    acc_sh.at[pl.ds(sid * _REG + (nz - 1) * 128,
                                    _REG - (nz - 1) * 128)])

    def lidx_of(dvec, base):
        local = dvec - base
        valid = (local >= 0) & (local < ob)
        return jnp.where(valid, local, ob) + sid * _REG

    def issue_gather(idx_ref, buf, sem):
        pltpu.async_copy(table_h.at[idx_ref], buf, sem)

    def wait_rows(buf, sem):
        pltpu.make_async_copy(table_h.at[src_s], buf, sem).wait()

    def issue_scatter(idx_ref, buf, sem):
        pltpu.async_copy(buf, acc_sh.at[idx_ref], sem, add=True)

    def wait_scatter(buf, sem):
        pltpu.make_async_copy(buf, acc_sh.at[gidx_s.at[0]], sem).wait()

    def per_buf(k, fn):
        @pl.when(k % 2 == 0)
        def _():
            fn(rows_v.at[0], gsem0, ssem0)

        @pl.when(k % 2 == 1)
        def _():
            fn(rows_v.at[1], gsem1, ssem1)

    def stage_for(kk_next):
        b2 = w + kk_next * 32
        st8 = (_extract(offa_v, b2) // 8) * 8

        def issue(slot):
            pltpu.async_copy(src_h.at[pl.ds(st8, capc * _CH)],
                             src_ws[slot], stsem)
            pltpu.async_copy(dst_h.at[pl.ds(st8, capc * _CH)],
                             dst_ws[slot], stsem)

        @pl.when(kk_next % 2 == 0)
        def _():
            issue(0)

        @pl.when(kk_next % 2 == 1)
        def _():
            issue(1)

    def wait_stage():
        for _ in range(2):
            pltpu.make_async_copy(src_h.at[pl.ds(0, capc * _CH)],
                                  src_w0, stsem).wait()

    def wait_ocopy():
        pltpu.make_async_copy(acc_sh.at[pl.ds(sid * _REG, ob)],
                              out_h.at[pl.ds(0, ob)], osem).wait()

    @pl.when(nblk > 0)
    def _():
        stage_for(0)

    def block_body(kk, carry):
        b = w + kk * 32
        start = _extract(offa_v, b)
        end = _extract(offb_v, b)
        start8 = (start // 8) * 8
        nch = (end - start8 + _CH - 1) // _CH
        kfast = jnp.minimum(nch, capc)
        base = b * ob

        wait_stage()

        @pl.when(kk + 1 < nblk)
        def _():
            stage_for(kk + 1)

        @pl.when(kk > 0)
        def _():
            wait_ocopy()
            for j in range(ob // 128):
                pltpu.sync_copy(zero_v,
                                acc_sh.at[pl.ds(sid * _REG + j * 128, 128)])

        def process(slot):
            def pre_body(ck, c2):
                for j in range(_CH // 16):
                    dvec = dst_ws[slot][pl.ds(ck * _CH + j * 16, 16)]
                    gidx2[ck, pl.ds(j * 16, 16)] = lidx_of(dvec, base)
                return c2

            lax.fori_loop(0, kfast, pre_body, 0)

            def start_fast(ck):
                per_buf(ck, lambda buf, gs, ss:
                        issue_gather(src_ws[slot].at[pl.ds(ck * _CH, _CH)],
                                     buf, gs))

            @pl.when(kfast > 0)
            def _():
                start_fast(0)

            def chunk_body(ck, c2):
                @pl.when(ck + 1 < kfast)
                def _():
                    @pl.when(ck >= 1)
                    def _():
                        per_buf(ck + 1,
                                lambda buf, gs, ss: wait_scatter(buf, ss))
                    start_fast(ck + 1)

                def fin(buf, gs, ss):
                    wait_rows(buf, gs)
                    issue_scatter(gidx2.at[ck], buf, ss)

                per_buf(ck, fin)
                return c2

            lax.fori_loop(0, kfast, chunk_body, 0)

            @pl.when(kfast >= 2)
            def _():
                per_buf(kfast - 2, lambda buf, gs, ss: wait_scatter(buf, ss))

            @pl.when(kfast >= 1)
            def _():
                per_buf(kfast - 1, lambda buf, gs, ss: wait_scatter(buf, ss))

        @pl.when(kk % 2 == 0)
        def _():
            process(0)

        @pl.when(kk % 2 == 1)
        def _():
            process(1)

        def slow_body(ck, c2):
            g0 = start8 + ck * _CH
            pltpu.sync_copy(src_h.at[pl.ds(g0, _CH)], src_s)
            pltpu.sync_copy(dst_h.at[pl.ds(g0, _CH)], dst_s)
            for j in range(_CH // 16):
                dvec = dst_s[pl.ds(j * 16, 16)]
                gidx_s[0, pl.ds(j * 16, 16)] = lidx_of(dvec, base)
            pltpu.async_copy(table_h.at[src_s], rows_v.at[0], gsem0)
            wait_rows(rows_v.at[0], gsem0)
            pltpu.async_copy(rows_v.at[0], acc_sh.at[gidx_s.at[0]], ssem0,
                             add=True)
            wait_scatter(rows_v.at[0], ssem0)
            return c2

        lax.fori_loop(kfast, nch, slow_body, 0)

        pltpu.async_copy(acc_sh.at[pl.ds(sid * _REG, ob)],
                         out_h.at[pl.ds(base, ob)], osem)
        return carry

    lax.fori_loop(0, nblk, block_body, 0)

    @pl.when(nblk > 0)
    def _():
        wait_ocopy()


def _seg_sum_gather(table, src, dst, num_out):
    n = src.shape[0]
    d = table.shape[1]
    ob = _OB
    nb = (num_out + ob - 1) // ob
    npad = nb * ob
    capc = min(_CAPC, 2 * max(1, -(-n // (nb * _CH))))
    pad = capc * _CH
    src_p = jnp.concatenate([src.astype(jnp.int32),
                             jnp.zeros((pad,), dtype=jnp.int32)])
    dst_p = jnp.concatenate([dst.astype(jnp.int32),
                             jnp.full((pad,), npad, dtype=jnp.int32)])
    cw = 1024
    nc = (n + cw - 1) // cw
    dmat = dst_p[:nc * cw].reshape(nc, cw)
    lead = dmat[:, 0]
    v = jnp.arange(nb + 1, dtype=jnp.int32) * ob
    j = jnp.maximum(
        jnp.searchsorted(lead, v, side='left', method='compare_all')
        .astype(jnp.int32) - 1, 0)
    rows = dmat[j]
    off = j * cw + jnp.sum(rows < v[:, None], axis=1, dtype=jnp.int32)
    padnb = 16 * ((nb + 31) // 16)
    filler = jnp.full((padnb - nb,), n, dtype=jnp.int32)
    offa = jnp.concatenate([off[:nb], filler])
    offb = jnp.concatenate([off[1:nb + 1], filler])
    zeros = jnp.zeros((128, d), dtype=jnp.float32)

    mesh = plsc.VectorSubcoreMesh(core_axis_name="c", subcore_axis_name="s")
    body = functools.partial(_sc_seg_body, nb, ob, d, capc)
    f = pl.kernel(
        body,
        out_type=jax.ShapeDtypeStruct((npad, d), jnp.float32),
        mesh=mesh,
        scratch_types=[
            pltpu.VMEM((padnb,), jnp.int32),
            pltpu.VMEM((padnb,), jnp.int32),
            pltpu.VMEM((capc * _CH,), jnp.int32),
            pltpu.VMEM((capc * _CH,), jnp.int32),
            pltpu.VMEM((capc * _CH,), jnp.int32),
            pltpu.VMEM((capc * _CH,), jnp.int32),
            pltpu.VMEM((capc, _CH), jnp.int32),
            pltpu.VMEM((_CH,), jnp.int32),
            pltpu.VMEM((_CH,), jnp.int32),
            pltpu.VMEM((1, _CH), jnp.int32),
            pltpu.VMEM((2, _CH, d), jnp.float32),
            pltpu.VMEM((128, d), jnp.float32),
            pltpu.VMEM_SHARED((16 * _REG, d), jnp.float32),
            pltpu.SemaphoreType.DMA,
            pltpu.SemaphoreType.DMA,
            pltpu.SemaphoreType.DMA,
            pltpu.SemaphoreType.DMA,
            pltpu.SemaphoreType.DMA,
            pltpu.SemaphoreType.DMA,
        ],
    )
    return f(table, src_p, dst_p, offa, offb, zeros)


def kernel(edge_rep, cycle_rep, e2c_src_5, e2c_dst_5, e2c_src_6, e2c_dst_6,
           c2c_src_5, c2c_dst_5, c2c_src_6, c2c_dst_6, c2e_src, c2e_dst,
           lin_w_5, lin_b_5, lin_w_6, lin_b_6,
           cmlp_w0, cmlp_b0, cmlp_w1, cmlp_b1, cmlp_w2, cmlp_b2,
           emlp_w0, emlp_b0, emlp_w1, emlp_b1):
    nE, d = edge_rep.shape
    r5 = c2c_src_5.shape[0]
    r6 = c2c_src_6.shape[0]

    w0a = cmlp_w0[:d]
    w0b = cmlp_w0[d:3 * d]
    w0c = cmlp_w0[3 * d:4 * d]
    w0d = cmlp_w0[4 * d:]
    m5 = lin_w_5 @ w0b + w0d
    m6 = lin_w_6 @ w0b + w0d
    b0_5 = cmlp_b0 + lin_b_5 @ w0b
    b0_6 = cmlp_b0 + lin_b_6 @ w0b

    e2c_src = jnp.concatenate([e2c_src_5, e2c_src_6]).astype(jnp.int32)
    e2c_dst = jnp.concatenate([e2c_dst_5, e2c_dst_6 + r5]).astype(jnp.int32)
    e2c = _seg_sum_gather(edge_rep, e2c_src, e2c_dst, r5 + r6)
    c2c_src = jnp.concatenate([c2c_src_5, c2c_src_6 + r5]).astype(jnp.int32)
    c2c_dst = jnp.concatenate([c2c_dst_5, c2c_dst_6 + r5]).astype(jnp.int32)
    s = _seg_sum_gather(e2c, c2c_src, c2c_dst, r5 + r6)

    blk = 8000
    bf = jnp.bfloat16
    w02 = jnp.stack([jnp.concatenate([w0a, m5, w0c], axis=0),
                     jnp.concatenate([w0a, m6, w0c], axis=0)]).astype(bf)
    cycle_out = _cycle_mlp(cycle_rep, e2c, s, w02,
                           jnp.stack([b0_5, b0_6]),
                           cmlp_w1.astype(bf), cmlp_b1, cmlp_w2.astype(bf),
                           cmlp_b2, blk=blk, nblk5=r5 // blk)

    c2e = _seg_sum_gather(cycle_out, c2e_src, c2e_dst, nE)

    edge_out = _edge_mlp(edge_rep, c2e, emlp_w0.astype(bf),
                         emlp_b0, emlp_w1.astype(bf), emlp_b1, blk=8000)
    return (edge_out, cycle_out)
